# baseline (speedup 1.0000x reference)
import jax
import jax.numpy as jnp
from jax.experimental import pallas as pl
from jax.experimental.pallas import tpu as pltpu


def _fused_kernel(adj_ref, x_ref, w_ref, bias_ref, out_ref):
    tmp = jnp.dot(adj_ref[...], x_ref[...], precision=jax.lax.Precision.DEFAULT, preferred_element_type=jnp.float32)
    acc = jnp.dot(tmp, w_ref[...], precision=jax.lax.Precision.DEFAULT, preferred_element_type=jnp.float32)
    out_ref[...] = acc + bias_ref[...]


def kernel(inputs, adj, Weight, Bias):
    n, d = inputs.shape
    bias2d = Bias.reshape(1, d)
    bm = 400
    grid = (n // bm,)
    out = pl.pallas_call(
        _fused_kernel,
        grid=grid,
        in_specs=[
            pl.BlockSpec((bm, n), lambda i: (i, 0)),
            pl.BlockSpec((n, d), lambda i: (0, 0)),
            pl.BlockSpec((d, d), lambda i: (0, 0)),
            pl.BlockSpec((1, d), lambda i: (0, 0)),
        ],
        out_specs=pl.BlockSpec((bm, d), lambda i: (i, 0)),
        out_shape=jax.ShapeDtypeStruct((n, d), jnp.float32),
        compiler_params=pltpu.CompilerParams(dimension_semantics=("arbitrary",), vmem_limit_bytes=63*1024*1024),
    )(adj, inputs, Weight, bias2d)
    return out
